# baseline (speedup 1.0000x reference)
import functools

import jax
import jax.numpy as jnp
from jax import lax
from jax.experimental import pallas as pl
from jax.experimental.pallas import tpu as pltpu
from jax.experimental.pallas import tpu_sc as plsc

NC = 2
NS = 16
CH = 64
DW = 16
F0 = 0.68
PB = 14


def _round_up(v, m):
    return (v + m - 1) // m * m


@functools.lru_cache(maxsize=None)
def _make_sc_segsum(n_pad, nch_a, nch_b, width, with_deg):
    rows_per_tile = n_pad // NS
    nch_max = max(nch_a, nch_b)
    mesh = plsc.VectorSubcoreMesh(
        core_axis_name="c", subcore_axis_name="s", num_cores=NC, num_subcores=NS
    )
    out_type = [jax.ShapeDtypeStruct((NC, n_pad, width), jnp.float32)]
    scratch = [
        pltpu.VMEM((nch_max, CH), jnp.int32),
        pltpu.VMEM((CH,), jnp.int32),
        pltpu.VMEM((CH,), jnp.int32),
        pltpu.VMEM((CH,), jnp.int32),
        pltpu.VMEM((CH,), jnp.int32),
        pltpu.VMEM((CH, width), jnp.float32),
        pltpu.VMEM((CH, width), jnp.float32),
        pltpu.VMEM_SHARED((n_pad, width), jnp.float32),
        pltpu.SemaphoreType.DMA,
        pltpu.SemaphoreType.DMA,
    ]
    if with_deg:
        out_type.append(jax.ShapeDtypeStruct((NC, n_pad, DW), jnp.float32))
        scratch += [
            pltpu.VMEM((CH, DW), jnp.float32),
            pltpu.VMEM_SHARED((n_pad, DW), jnp.float32),
        ]

    @functools.partial(
        pl.kernel,
        out_type=out_type,
        mesh=mesh,
        scratch_types=scratch,
        compiler_params=pltpu.CompilerParams(use_tc_tiling_on_sc=False),
    )
    def sc_segsum(table_hbm, pk_hbm, zeros_hbm, *rest):
        if with_deg:
            (onehot_hbm, zdeg_hbm, out_hbm, deg_hbm,
             pk_v, src_a, dst_a, src_b, dst_b, rows_a, rows_b, acc_sh,
             sem_a, sem_b, ones_v, deg_sh) = rest
        else:
            (out_hbm,
             pk_v, src_a, dst_a, src_b, dst_b, rows_a, rows_b, acc_sh,
             sem_a, sem_b) = rest
        cid = lax.axis_index("c")
        sid = lax.axis_index("s")
        row0 = sid * rows_per_tile
        base = jnp.where(cid == 0, sid * nch_a, NS * nch_a + sid * nch_b)
        npairs = jnp.where(cid == 0, nch_a // 2, nch_b // 2)
        nmy = jnp.where(cid == 0, nch_a, nch_b)
        pltpu.sync_copy(zeros_hbm, acc_sh.at[pl.ds(row0, rows_per_tile)])
        pltpu.sync_copy(pk_hbm.at[pl.ds(base, nch_max)], pk_v)
        if with_deg:
            pltpu.sync_copy(onehot_hbm, ones_v)
            pltpu.sync_copy(zdeg_hbm, deg_sh.at[pl.ds(row0, rows_per_tile)])
        plsc.subcore_barrier()

        def unpack(j, sbuf, dbuf):
            row = pk_v.at[j]
            for k in range(CH // 16):
                v = row[pl.ds(16 * k, 16)]
                sbuf[pl.ds(16 * k, 16)] = lax.bitwise_and(v, 2 ** PB - 1)
                dbuf[pl.ds(16 * k, 16)] = lax.shift_right_logical(v, PB)

        unpack(0, src_a, dst_a)
        pltpu.async_copy(table_hbm.at[src_a], rows_a, sem_a)

        def body(p, carry):
            j0 = 2 * p
            unpack(j0 + 1, src_b, dst_b)
            pltpu.async_copy(table_hbm.at[src_b], rows_b, sem_b)
            pltpu.make_async_copy(table_hbm.at[src_a], rows_a, sem_a).wait()
            pltpu.sync_copy(rows_a, acc_sh.at[dst_a], add=True)
            if with_deg:
                pltpu.sync_copy(ones_v, deg_sh.at[dst_a], add=True)

            @pl.when(j0 + 2 < nmy)
            def _():
                unpack(j0 + 2, src_a, dst_a)
                pltpu.async_copy(table_hbm.at[src_a], rows_a, sem_a)

            pltpu.make_async_copy(table_hbm.at[src_b], rows_b, sem_b).wait()
            pltpu.sync_copy(rows_b, acc_sh.at[dst_b], add=True)
            if with_deg:
                pltpu.sync_copy(ones_v, deg_sh.at[dst_b], add=True)
            return carry

        lax.fori_loop(0, npairs, body, 0)
        plsc.subcore_barrier()
        pltpu.sync_copy(acc_sh.at[pl.ds(row0, rows_per_tile)],
                        out_hbm.at[cid, pl.ds(row0, rows_per_tile)])
        if with_deg:
            pltpu.sync_copy(deg_sh.at[pl.ds(row0, rows_per_tile)],
                            deg_hbm.at[cid, pl.ds(row0, rows_per_tile)])

    return sc_segsum


def _tc_mid_body(x_ref, acc_ref, deg_ref, ws0_ref, wn0_ref, b0_ref, wn1_ref,
                 ws1_ref, y1_ref, hs_ref, *, c1):
    deg = jnp.clip(deg_ref[0][:, 0:1] + deg_ref[1][:, 0:1], 1.0, None)
    mean = (acc_ref[0] + acc_ref[1]) / deg
    h = jnp.dot(x_ref[...], ws0_ref[...], preferred_element_type=jnp.float32)
    h = h + jnp.dot(mean, wn0_ref[...], preferred_element_type=jnp.float32)
    h = jnp.maximum(h + b0_ref[...], 0.0)
    y1 = jnp.dot(h, wn1_ref[...], preferred_element_type=jnp.float32)
    col = lax.broadcasted_iota(jnp.int32, y1.shape, 1)
    y1_ref[...] = jnp.where(col == c1 - 1, 1.0, y1)
    hs_ref[...] = jnp.dot(h, ws1_ref[...], preferred_element_type=jnp.float32)


def _tc_out_body(hs_ref, acc_ref, b1_ref, o_ref, *, ccol):
    a = acc_ref[0] + acc_ref[1]
    deg = jnp.clip(a[:, ccol:ccol + 1], 1.0, None)
    o_ref[...] = hs_ref[...] + a / deg + b1_ref[...]


def _split_chunks(e):
    t0 = -(-e // CH)
    nch_a = max(2, _round_up(int(t0 * F0 / NS + 0.5), 2))
    rem = max(e - NS * nch_a * CH, 0)
    nch_b = max(2, _round_up(-(-rem // (NS * CH)), 2))
    return nch_a, nch_b


def _prep_edges(edge_index, nch_a, nch_b, dump_row):
    nch_max = max(nch_a, nch_b)
    e_pad = NS * (nch_a + nch_b) * CH
    t_alloc = e_pad // CH + max(nch_a - nch_b, 0)
    src = edge_index[0].astype(jnp.int32)
    dst = edge_index[1].astype(jnp.int32)
    pk = dst * (2 ** PB) + src
    pad = t_alloc * CH - pk.shape[0]
    if pad:
        pk = jnp.concatenate([pk, jnp.full((pad,), dump_row * (2 ** PB), jnp.int32)])
    return pk.reshape(t_alloc, CH), nch_max


def kernel(x, edge_index0, edge_index1, W_self0, W_neigh0, b0, W_self1, W_neigh1, b1):
    n, d = x.shape
    h_dim = W_self0.shape[1]
    c = W_self1.shape[1]
    e = edge_index0.shape[1]

    c1 = _round_up(c + 1, 16)
    n_pad = _round_up(n + 1, 128)
    rows_per_tile = n_pad // NS
    nch_a, nch_b = _split_chunks(e)

    pk0, _ = _prep_edges(edge_index0, nch_a, nch_b, n)
    pk1, _ = _prep_edges(edge_index1, nch_a, nch_b, n)

    zeros0 = jnp.zeros((rows_per_tile, d), jnp.float32)
    zeros1 = jnp.zeros((rows_per_tile, c1), jnp.float32)
    zdeg = jnp.zeros((rows_per_tile, DW), jnp.float32)
    onehot = jnp.zeros((CH, DW), jnp.float32).at[:, 0].set(1.0)

    b0r = b0.reshape(1, h_dim)
    b1p = jnp.concatenate([b1, jnp.zeros((c1 - c,), jnp.float32)]).reshape(1, c1)
    wn1p = jnp.pad(W_neigh1, ((0, 0), (0, c1 - c)))
    ws1p = jnp.pad(W_self1, ((0, 0), (0, c1 - c)))

    acc0, deg0 = _make_sc_segsum(n_pad, nch_a, nch_b, d, True)(
        x, pk0, zeros0, onehot, zdeg)

    r = 1000
    grid = (n // r,)
    y1p, hs = pl.pallas_call(
        functools.partial(_tc_mid_body, c1=c1),
        grid=grid,
        in_specs=[
            pl.BlockSpec((r, d), lambda i: (i, 0)),
            pl.BlockSpec((NC, r, d), lambda i: (0, i, 0)),
            pl.BlockSpec((NC, r, DW), lambda i: (0, i, 0)),
            pl.BlockSpec((d, h_dim), lambda i: (0, 0)),
            pl.BlockSpec((d, h_dim), lambda i: (0, 0)),
            pl.BlockSpec((1, h_dim), lambda i: (0, 0)),
            pl.BlockSpec((h_dim, c1), lambda i: (0, 0)),
            pl.BlockSpec((h_dim, c1), lambda i: (0, 0)),
        ],
        out_specs=[pl.BlockSpec((r, c1), lambda i: (i, 0)),
                   pl.BlockSpec((r, c1), lambda i: (i, 0))],
        out_shape=[jax.ShapeDtypeStruct((n, c1), jnp.float32),
                   jax.ShapeDtypeStruct((n, c1), jnp.float32)],
    )(x, acc0, deg0, W_self0, W_neigh0, b0r, wn1p, ws1p)

    (acc1,) = _make_sc_segsum(n_pad, nch_a, nch_b, c1, False)(y1p, pk1, zeros1)

    out = pl.pallas_call(
        functools.partial(_tc_out_body, ccol=c),
        grid=grid,
        in_specs=[
            pl.BlockSpec((r, c1), lambda i: (i, 0)),
            pl.BlockSpec((NC, r, c1), lambda i: (0, i, 0)),
            pl.BlockSpec((1, c1), lambda i: (0, 0)),
        ],
        out_specs=pl.BlockSpec((r, c1), lambda i: (i, 0)),
        out_shape=jax.ShapeDtypeStruct((n, c1), jnp.float32),
    )(hs, acc1, b1p)

    return out[:, :c]


# speedup vs baseline: 1.0599x; 1.0599x over previous
import functools

import jax
import jax.numpy as jnp
from jax import lax
from jax.experimental import pallas as pl
from jax.experimental.pallas import tpu as pltpu
from jax.experimental.pallas import tpu_sc as plsc

NC = 2
NS = 16
CH = 64
DW = 16
F0 = 0.58
PB = 14


def _round_up(v, m):
    return (v + m - 1) // m * m


@functools.lru_cache(maxsize=None)
def _make_sc_segsum(n_pad, nch_a, nch_b, width, with_deg):
    rows_per_tile = n_pad // NS
    nch_max = max(nch_a, nch_b)
    mesh = plsc.VectorSubcoreMesh(
        core_axis_name="c", subcore_axis_name="s", num_cores=NC, num_subcores=NS
    )
    out_type = [jax.ShapeDtypeStruct((NC, n_pad, width), jnp.float32)]
    scratch = [
        pltpu.VMEM((nch_max, CH), jnp.int32),
        pltpu.VMEM((CH,), jnp.int32),
        pltpu.VMEM((CH,), jnp.int32),
        pltpu.VMEM((CH,), jnp.int32),
        pltpu.VMEM((CH,), jnp.int32),
        pltpu.VMEM((CH, width), jnp.float32),
        pltpu.VMEM((CH, width), jnp.float32),
        pltpu.VMEM_SHARED((n_pad, width), jnp.float32),
        pltpu.SemaphoreType.DMA,
        pltpu.SemaphoreType.DMA,
    ]
    if with_deg:
        out_type.append(jax.ShapeDtypeStruct((NC, n_pad, DW), jnp.float32))
        scratch += [
            pltpu.VMEM((CH, DW), jnp.float32),
            pltpu.VMEM_SHARED((n_pad, DW), jnp.float32),
        ]

    @functools.partial(
        pl.kernel,
        out_type=out_type,
        mesh=mesh,
        scratch_types=scratch,
        compiler_params=pltpu.CompilerParams(use_tc_tiling_on_sc=False),
    )
    def sc_segsum(table_hbm, pk_hbm, zeros_hbm, *rest):
        if with_deg:
            (onehot_hbm, zdeg_hbm, out_hbm, deg_hbm,
             pk_v, src_a, dst_a, src_b, dst_b, rows_a, rows_b, acc_sh,
             sem_a, sem_b, ones_v, deg_sh) = rest
        else:
            (out_hbm,
             pk_v, src_a, dst_a, src_b, dst_b, rows_a, rows_b, acc_sh,
             sem_a, sem_b) = rest
        cid = lax.axis_index("c")
        sid = lax.axis_index("s")
        row0 = sid * rows_per_tile
        base = jnp.where(cid == 0, sid * nch_a, NS * nch_a + sid * nch_b)
        npairs = jnp.where(cid == 0, nch_a // 2, nch_b // 2)
        nmy = jnp.where(cid == 0, nch_a, nch_b)
        pltpu.sync_copy(zeros_hbm, acc_sh.at[pl.ds(row0, rows_per_tile)])
        pltpu.sync_copy(pk_hbm.at[pl.ds(base, nch_max)], pk_v)
        if with_deg:
            pltpu.sync_copy(onehot_hbm, ones_v)
            pltpu.sync_copy(zdeg_hbm, deg_sh.at[pl.ds(row0, rows_per_tile)])
        plsc.subcore_barrier()

        def unpack(j, sbuf, dbuf):
            row = pk_v.at[j]
            for k in range(CH // 16):
                v = row[pl.ds(16 * k, 16)]
                sbuf[pl.ds(16 * k, 16)] = lax.bitwise_and(v, 2 ** PB - 1)
                dbuf[pl.ds(16 * k, 16)] = lax.shift_right_logical(v, PB)

        unpack(0, src_a, dst_a)
        pltpu.async_copy(table_hbm.at[src_a], rows_a, sem_a)

        def body(p, carry):
            j0 = 2 * p
            unpack(j0 + 1, src_b, dst_b)
            pltpu.async_copy(table_hbm.at[src_b], rows_b, sem_b)
            pltpu.make_async_copy(table_hbm.at[src_a], rows_a, sem_a).wait()
            pltpu.sync_copy(rows_a, acc_sh.at[dst_a], add=True)
            if with_deg:
                pltpu.sync_copy(ones_v, deg_sh.at[dst_a], add=True)

            @pl.when(j0 + 2 < nmy)
            def _():
                unpack(j0 + 2, src_a, dst_a)
                pltpu.async_copy(table_hbm.at[src_a], rows_a, sem_a)

            pltpu.make_async_copy(table_hbm.at[src_b], rows_b, sem_b).wait()
            pltpu.sync_copy(rows_b, acc_sh.at[dst_b], add=True)
            if with_deg:
                pltpu.sync_copy(ones_v, deg_sh.at[dst_b], add=True)
            return carry

        lax.fori_loop(0, npairs, body, 0)
        plsc.subcore_barrier()
        pltpu.sync_copy(acc_sh.at[pl.ds(row0, rows_per_tile)],
                        out_hbm.at[cid, pl.ds(row0, rows_per_tile)])
        if with_deg:
            pltpu.sync_copy(deg_sh.at[pl.ds(row0, rows_per_tile)],
                            deg_hbm.at[cid, pl.ds(row0, rows_per_tile)])

    return sc_segsum


def _tc_mid_body(x_ref, acc_ref, deg_ref, ws0_ref, wn0_ref, b0_ref, wn1_ref,
                 ws1_ref, y1_ref, hs_ref, *, c1):
    deg = jnp.clip(deg_ref[0][:, 0:1] + deg_ref[1][:, 0:1], 1.0, None)
    mean = (acc_ref[0] + acc_ref[1]) / deg
    h = jnp.dot(x_ref[...], ws0_ref[...], preferred_element_type=jnp.float32)
    h = h + jnp.dot(mean, wn0_ref[...], preferred_element_type=jnp.float32)
    h = jnp.maximum(h + b0_ref[...], 0.0)
    y1 = jnp.dot(h, wn1_ref[...], preferred_element_type=jnp.float32)
    col = lax.broadcasted_iota(jnp.int32, y1.shape, 1)
    y1_ref[...] = jnp.where(col == c1 - 1, 1.0, y1)
    hs_ref[...] = jnp.dot(h, ws1_ref[...], preferred_element_type=jnp.float32)


def _tc_out_body(hs_ref, acc_ref, b1_ref, o_ref, *, ccol):
    a = acc_ref[0] + acc_ref[1]
    deg = jnp.clip(a[:, ccol:ccol + 1], 1.0, None)
    o_ref[...] = hs_ref[...] + a / deg + b1_ref[...]


def _split_chunks(e):
    t0 = -(-e // CH)
    nch_a = max(2, _round_up(int(t0 * F0 / NS + 0.5), 2))
    rem = max(e - NS * nch_a * CH, 0)
    nch_b = max(2, _round_up(-(-rem // (NS * CH)), 2))
    return nch_a, nch_b


def _prep_edges(edge_index, nch_a, nch_b, dump_row):
    nch_max = max(nch_a, nch_b)
    e_pad = NS * (nch_a + nch_b) * CH
    t_alloc = e_pad // CH + max(nch_a - nch_b, 0)
    src = edge_index[0].astype(jnp.int32)
    dst = edge_index[1].astype(jnp.int32)
    pk = dst * (2 ** PB) + src
    pad = t_alloc * CH - pk.shape[0]
    if pad:
        pk = jnp.concatenate([pk, jnp.full((pad,), dump_row * (2 ** PB), jnp.int32)])
    return pk.reshape(t_alloc, CH), nch_max


def kernel(x, edge_index0, edge_index1, W_self0, W_neigh0, b0, W_self1, W_neigh1, b1):
    n, d = x.shape
    h_dim = W_self0.shape[1]
    c = W_self1.shape[1]
    e = edge_index0.shape[1]

    c1 = _round_up(c + 1, 16)
    n_pad = _round_up(n + 1, 128)
    rows_per_tile = n_pad // NS
    nch_a, nch_b = _split_chunks(e)

    pk0, _ = _prep_edges(edge_index0, nch_a, nch_b, n)
    pk1, _ = _prep_edges(edge_index1, nch_a, nch_b, n)

    zeros0 = jnp.zeros((rows_per_tile, d), jnp.float32)
    zeros1 = jnp.zeros((rows_per_tile, c1), jnp.float32)
    zdeg = jnp.zeros((rows_per_tile, DW), jnp.float32)
    onehot = jnp.zeros((CH, DW), jnp.float32).at[:, 0].set(1.0)

    b0r = b0.reshape(1, h_dim)
    b1p = jnp.concatenate([b1, jnp.zeros((c1 - c,), jnp.float32)]).reshape(1, c1)
    wn1p = jnp.pad(W_neigh1, ((0, 0), (0, c1 - c)))
    ws1p = jnp.pad(W_self1, ((0, 0), (0, c1 - c)))

    acc0, deg0 = _make_sc_segsum(n_pad, nch_a, nch_b, d, True)(
        x, pk0, zeros0, onehot, zdeg)

    r = 1000
    grid = (n // r,)
    y1p, hs = pl.pallas_call(
        functools.partial(_tc_mid_body, c1=c1),
        grid=grid,
        in_specs=[
            pl.BlockSpec((r, d), lambda i: (i, 0)),
            pl.BlockSpec((NC, r, d), lambda i: (0, i, 0)),
            pl.BlockSpec((NC, r, DW), lambda i: (0, i, 0)),
            pl.BlockSpec((d, h_dim), lambda i: (0, 0)),
            pl.BlockSpec((d, h_dim), lambda i: (0, 0)),
            pl.BlockSpec((1, h_dim), lambda i: (0, 0)),
            pl.BlockSpec((h_dim, c1), lambda i: (0, 0)),
            pl.BlockSpec((h_dim, c1), lambda i: (0, 0)),
        ],
        out_specs=[pl.BlockSpec((r, c1), lambda i: (i, 0)),
                   pl.BlockSpec((r, c1), lambda i: (i, 0))],
        out_shape=[jax.ShapeDtypeStruct((n, c1), jnp.float32),
                   jax.ShapeDtypeStruct((n, c1), jnp.float32)],
    )(x, acc0, deg0, W_self0, W_neigh0, b0r, wn1p, ws1p)

    (acc1,) = _make_sc_segsum(n_pad, nch_a, nch_b, c1, False)(y1p, pk1, zeros1)

    out = pl.pallas_call(
        functools.partial(_tc_out_body, ccol=c),
        grid=grid,
        in_specs=[
            pl.BlockSpec((r, c1), lambda i: (i, 0)),
            pl.BlockSpec((NC, r, c1), lambda i: (0, i, 0)),
            pl.BlockSpec((1, c1), lambda i: (0, 0)),
        ],
        out_specs=pl.BlockSpec((r, c1), lambda i: (i, 0)),
        out_shape=jax.ShapeDtypeStruct((n, c1), jnp.float32),
    )(hs, acc1, b1p)

    return out[:, :c]


# speedup vs baseline: 1.1315x; 1.0676x over previous
import functools

import jax
import jax.numpy as jnp
from jax import lax
from jax.experimental import pallas as pl
from jax.experimental.pallas import tpu as pltpu
from jax.experimental.pallas import tpu_sc as plsc

NC = 2
NS = 16
CH = 64
DW = 16
F0 = 0.62
PB = 14


def _round_up(v, m):
    return (v + m - 1) // m * m


@functools.lru_cache(maxsize=None)
def _make_sc_segsum(n_pad, nch_a, nch_b, width, with_deg):
    rows_per_tile = n_pad // NS
    nch_max = max(nch_a, nch_b)
    mesh = plsc.VectorSubcoreMesh(
        core_axis_name="c", subcore_axis_name="s", num_cores=NC, num_subcores=NS
    )
    out_type = [jax.ShapeDtypeStruct((NC, n_pad, width), jnp.float32)]
    scratch = [
        pltpu.VMEM((nch_max, CH), jnp.int32),
        pltpu.VMEM((CH,), jnp.int32),
        pltpu.VMEM((CH,), jnp.int32),
        pltpu.VMEM((CH,), jnp.int32),
        pltpu.VMEM((CH,), jnp.int32),
        pltpu.VMEM((CH, width), jnp.float32),
        pltpu.VMEM((CH, width), jnp.float32),
        pltpu.VMEM_SHARED((n_pad, width), jnp.float32),
        pltpu.SemaphoreType.DMA,
        pltpu.SemaphoreType.DMA,
    ]
    if with_deg:
        out_type.append(jax.ShapeDtypeStruct((NC, n_pad, DW), jnp.float32))
        scratch += [
            pltpu.VMEM((CH, DW), jnp.float32),
            pltpu.VMEM_SHARED((n_pad, DW), jnp.float32),
        ]

    @functools.partial(
        pl.kernel,
        out_type=out_type,
        mesh=mesh,
        scratch_types=scratch,
        compiler_params=pltpu.CompilerParams(use_tc_tiling_on_sc=False),
    )
    def sc_segsum(table_hbm, pk_hbm, zeros_hbm, *rest):
        if with_deg:
            (onehot_hbm, zdeg_hbm, out_hbm, deg_hbm,
             pk_v, src_a, dst_a, src_b, dst_b, rows_a, rows_b, acc_sh,
             sem_a, sem_b, ones_v, deg_sh) = rest
        else:
            (out_hbm,
             pk_v, src_a, dst_a, src_b, dst_b, rows_a, rows_b, acc_sh,
             sem_a, sem_b) = rest
        cid = lax.axis_index("c")
        sid = lax.axis_index("s")
        row0 = sid * rows_per_tile
        base = jnp.where(cid == 0, sid * nch_a, NS * nch_a + sid * nch_b)
        npairs = jnp.where(cid == 0, nch_a // 2, nch_b // 2)
        nmy = jnp.where(cid == 0, nch_a, nch_b)
        pltpu.sync_copy(zeros_hbm, acc_sh.at[pl.ds(row0, rows_per_tile)])
        pltpu.sync_copy(pk_hbm.at[pl.ds(base, nch_max)], pk_v)
        if with_deg:
            pltpu.sync_copy(onehot_hbm, ones_v)
            pltpu.sync_copy(zdeg_hbm, deg_sh.at[pl.ds(row0, rows_per_tile)])
        plsc.subcore_barrier()

        def unpack(j, sbuf, dbuf):
            row = pk_v.at[j]
            for k in range(CH // 16):
                v = row[pl.ds(16 * k, 16)]
                sbuf[pl.ds(16 * k, 16)] = lax.bitwise_and(v, 2 ** PB - 1)
                dbuf[pl.ds(16 * k, 16)] = lax.shift_right_logical(v, PB)

        unpack(0, src_a, dst_a)
        pltpu.async_copy(table_hbm.at[src_a], rows_a, sem_a)

        def body(p, carry):
            j0 = 2 * p
            unpack(j0 + 1, src_b, dst_b)
            pltpu.async_copy(table_hbm.at[src_b], rows_b, sem_b)
            pltpu.make_async_copy(table_hbm.at[src_a], rows_a, sem_a).wait()
            pltpu.sync_copy(rows_a, acc_sh.at[dst_a], add=True)
            if with_deg:
                pltpu.sync_copy(ones_v, deg_sh.at[dst_a], add=True)

            @pl.when(j0 + 2 < nmy)
            def _():
                unpack(j0 + 2, src_a, dst_a)
                pltpu.async_copy(table_hbm.at[src_a], rows_a, sem_a)

            pltpu.make_async_copy(table_hbm.at[src_b], rows_b, sem_b).wait()
            pltpu.sync_copy(rows_b, acc_sh.at[dst_b], add=True)
            if with_deg:
                pltpu.sync_copy(ones_v, deg_sh.at[dst_b], add=True)
            return carry

        lax.fori_loop(0, npairs, body, 0)
        plsc.subcore_barrier()
        pltpu.sync_copy(acc_sh.at[pl.ds(row0, rows_per_tile)],
                        out_hbm.at[cid, pl.ds(row0, rows_per_tile)])
        if with_deg:
            pltpu.sync_copy(deg_sh.at[pl.ds(row0, rows_per_tile)],
                            deg_hbm.at[cid, pl.ds(row0, rows_per_tile)])

    return sc_segsum


def _tc_mid_body(x_ref, acc_ref, deg_ref, ws0_ref, wn0_ref, b0_ref, wn1_ref,
                 ws1_ref, y1_ref, hs_ref, *, c1):
    deg = jnp.clip(deg_ref[0][:, 0:1] + deg_ref[1][:, 0:1], 1.0, None)
    mean = (acc_ref[0] + acc_ref[1]) / deg
    h = jnp.dot(x_ref[...], ws0_ref[...], preferred_element_type=jnp.float32)
    h = h + jnp.dot(mean, wn0_ref[...], preferred_element_type=jnp.float32)
    h = jnp.maximum(h + b0_ref[...], 0.0)
    y1 = jnp.dot(h, wn1_ref[...], preferred_element_type=jnp.float32)
    col = lax.broadcasted_iota(jnp.int32, y1.shape, 1)
    y1_ref[...] = jnp.where(col == c1 - 1, 1.0, y1)
    hs_ref[...] = jnp.dot(h, ws1_ref[...], preferred_element_type=jnp.float32)


def _tc_out_body(hs_ref, acc_ref, b1_ref, o_ref, *, ccol):
    a = acc_ref[0] + acc_ref[1]
    deg = jnp.clip(a[:, ccol:ccol + 1], 1.0, None)
    o_ref[...] = hs_ref[...] + a / deg + b1_ref[...]


def _split_chunks(e):
    t0 = -(-e // CH)
    nch_a = max(2, _round_up(int(t0 * F0 / NS + 0.5), 2))
    rem = max(e - NS * nch_a * CH, 0)
    nch_b = max(2, _round_up(-(-rem // (NS * CH)), 2))
    return nch_a, nch_b


def _prep_edges(edge_index, nch_a, nch_b, dump_row):
    nch_max = max(nch_a, nch_b)
    e_pad = NS * (nch_a + nch_b) * CH
    t_alloc = e_pad // CH + max(nch_a - nch_b, 0)
    src = edge_index[0].astype(jnp.int32)
    dst = edge_index[1].astype(jnp.int32)
    pk = dst * (2 ** PB) + src
    pad = t_alloc * CH - pk.shape[0]
    if pad:
        pk = jnp.concatenate([pk, jnp.full((pad,), dump_row * (2 ** PB), jnp.int32)])
    return pk.reshape(t_alloc, CH), nch_max


def kernel(x, edge_index0, edge_index1, W_self0, W_neigh0, b0, W_self1, W_neigh1, b1):
    n, d = x.shape
    h_dim = W_self0.shape[1]
    c = W_self1.shape[1]
    e = edge_index0.shape[1]

    c1 = _round_up(c + 1, 16)
    n_pad = _round_up(n + 1, 128)
    rows_per_tile = n_pad // NS
    nch_a, nch_b = _split_chunks(e)

    pk0, _ = _prep_edges(edge_index0, nch_a, nch_b, n)
    pk1, _ = _prep_edges(edge_index1, nch_a, nch_b, n)

    zeros0 = jnp.zeros((rows_per_tile, d), jnp.float32)
    zeros1 = jnp.zeros((rows_per_tile, c1), jnp.float32)
    zdeg = jnp.zeros((rows_per_tile, DW), jnp.float32)
    onehot = jnp.zeros((CH, DW), jnp.float32).at[:, 0].set(1.0)

    b0r = b0.reshape(1, h_dim)
    b1p = jnp.concatenate([b1, jnp.zeros((c1 - c,), jnp.float32)]).reshape(1, c1)
    wn1p = jnp.pad(W_neigh1, ((0, 0), (0, c1 - c)))
    ws1p = jnp.pad(W_self1, ((0, 0), (0, c1 - c)))

    acc0, deg0 = _make_sc_segsum(n_pad, nch_a, nch_b, d, True)(
        x, pk0, zeros0, onehot, zdeg)

    r = 1000
    grid = (n // r,)
    y1p, hs = pl.pallas_call(
        functools.partial(_tc_mid_body, c1=c1),
        grid=grid,
        in_specs=[
            pl.BlockSpec((r, d), lambda i: (i, 0)),
            pl.BlockSpec((NC, r, d), lambda i: (0, i, 0)),
            pl.BlockSpec((NC, r, DW), lambda i: (0, i, 0)),
            pl.BlockSpec((d, h_dim), lambda i: (0, 0)),
            pl.BlockSpec((d, h_dim), lambda i: (0, 0)),
            pl.BlockSpec((1, h_dim), lambda i: (0, 0)),
            pl.BlockSpec((h_dim, c1), lambda i: (0, 0)),
            pl.BlockSpec((h_dim, c1), lambda i: (0, 0)),
        ],
        out_specs=[pl.BlockSpec((r, c1), lambda i: (i, 0)),
                   pl.BlockSpec((r, c1), lambda i: (i, 0))],
        out_shape=[jax.ShapeDtypeStruct((n, c1), jnp.float32),
                   jax.ShapeDtypeStruct((n, c1), jnp.float32)],
    )(x, acc0, deg0, W_self0, W_neigh0, b0r, wn1p, ws1p)

    (acc1,) = _make_sc_segsum(n_pad, nch_a, nch_b, c1, False)(y1p, pk1, zeros1)

    out = pl.pallas_call(
        functools.partial(_tc_out_body, ccol=c),
        grid=grid,
        in_specs=[
            pl.BlockSpec((r, c1), lambda i: (i, 0)),
            pl.BlockSpec((NC, r, c1), lambda i: (0, i, 0)),
            pl.BlockSpec((1, c1), lambda i: (0, 0)),
        ],
        out_specs=pl.BlockSpec((r, c1), lambda i: (i, 0)),
        out_shape=jax.ShapeDtypeStruct((n, c1), jnp.float32),
    )(hs, acc1, b1p)

    return out[:, :c]
